# baseline (speedup 1.0000x reference)
import functools
import math

import jax
import jax.numpy as jnp
from jax import lax
from jax.experimental import pallas as pl
from jax.experimental.pallas import tpu as pltpu
from jax.experimental.pallas import tpu_sc as plsc

H = 769
HP = 784
HPB = 800
NBB = HPB // 32
SCALE = 1.0 / math.sqrt(H)
NC, NS = 2, 16
NW = NC * NS
R = 2048
RS = R // NS
ZR = 16

_SC_PARAMS = pltpu.CompilerParams(use_tc_tiling_on_sc=False,
                                  needs_layout_passes=False)


def _ceil_mult(x, m):
    return ((x + m - 1) // m) * m


def _mm(a, w, b, act="none", denom_mod=False):
    n, k = a.shape
    m = w.shape[1]
    bn = 512
    grid = (pl.cdiv(n, bn),)

    def body(a_ref, w_ref, b_ref, o_ref):
        acc = jnp.dot(a_ref[...], w_ref[...], preferred_element_type=jnp.float32)
        acc = acc + b_ref[...]
        if act == "relu":
            acc = jnp.maximum(acc, 0.0)
        if denom_mod:
            col = lax.broadcasted_iota(jnp.int32, acc.shape, 1)
            acc = jnp.where(col % HP == H, 1.0, acc)
        o_ref[...] = acc

    return pl.pallas_call(
        body,
        grid=grid,
        in_specs=[
            pl.BlockSpec((bn, k), lambda i: (i, 0)),
            pl.BlockSpec((k, m), lambda i: (0, 0)),
            pl.BlockSpec((1, m), lambda i: (0, 0)),
        ],
        out_specs=pl.BlockSpec((bn, m), lambda i: (i, 0)),
        out_shape=jax.ShapeDtypeStruct((n, m), jnp.float32),
    )(a, w, b.reshape(1, m))


def _out_stage(acc0, acc1, hprev, w, b, sk):
    n = hprev.shape[0]
    bn = 512
    grid = (pl.cdiv(n, bn),)

    def body(a0_ref, a1_ref, h_ref, w_ref, b_ref, sk_ref, o_ref):
        s = a0_ref[...].astype(jnp.float32) + a1_ref[...].astype(jnp.float32)
        denom = s[:, H:H + 1]
        g = s * (1.0 / (denom + 1e-16))
        g = g * 0.5 * (1.0 + lax.erf(g * (1.0 / math.sqrt(2.0))))
        o = jnp.dot(g, w_ref[...], preferred_element_type=jnp.float32) + b_ref[...]
        skv = sk_ref[0, 0]
        o_ref[...] = jnp.maximum(skv * o + (1.0 - skv) * h_ref[...], 0.0)

    return pl.pallas_call(
        body,
        grid=grid,
        in_specs=[
            pl.BlockSpec((bn, HPB), lambda i: (i, 0)),
            pl.BlockSpec((bn, HPB), lambda i: (i, 0)),
            pl.BlockSpec((bn, HP), lambda i: (i, 0)),
            pl.BlockSpec((HPB, HP), lambda i: (0, 0)),
            pl.BlockSpec((1, HP), lambda i: (0, 0)),
            pl.BlockSpec((1, 1), lambda i: (0, 0)),
        ],
        out_specs=pl.BlockSpec((bn, HP), lambda i: (i, 0)),
        out_shape=jax.ShapeDtypeStruct((n, HP), jnp.float32),
    )(acc0, acc1, hprev, w, b.reshape(1, HP), sk.reshape(1, 1))


@functools.lru_cache(maxsize=None)
def _make_edge(npad, nk, epad, ecat):
    ew = epad // NW
    nblk = ew // 32
    npasses = npad // R
    mcap = ew + 48
    mesh = plsc.VectorSubcoreMesh(core_axis_name="c", subcore_axis_name="s",
                                  num_cores=NC, num_subcores=NS)

    @functools.partial(
        pl.kernel,
        out_type=jax.ShapeDtypeStruct((NC, npad, HPB), jnp.bfloat16),
        mesh=mesh,
        compiler_params=_SC_PARAMS,
        scratch_types=[
            pltpu.VMEM((ew,), jnp.int32),
            pltpu.VMEM((ew,), jnp.int32),
            pltpu.VMEM((ew,), jnp.float32),
            pltpu.VMEM((mcap,), jnp.int32),
            pltpu.VMEM((mcap,), jnp.int32),
            pltpu.VMEM((mcap,), jnp.float32),
            pltpu.VMEM((2, 32), jnp.int32),
            pltpu.VMEM((2, 32), jnp.int32),
            pltpu.VMEM((2, 32, HPB), jnp.bfloat16),
            pltpu.VMEM((2, 32, HPB), jnp.bfloat16),
            pltpu.VMEM((ZR, HPB), jnp.bfloat16),
            pltpu.VMEM_SHARED((R, HPB), jnp.bfloat16),
            pltpu.SemaphoreType.DMA((2,)),
            pltpu.SemaphoreType.DMA((2,)),
        ],
    )
    def s12(q_hbm, k_hbm, v_hbm, dst_hbm, src_hbm, out_hbm,
            edst, esrc, wloc, mdl, msrc, mw, dstg, sstg, qrows, krows,
            zbuf, accsh, qsem, ksem):
        cc = lax.axis_index("c")
        sid = lax.axis_index("s")
        wid = sid * NC + cc
        base = wid * ew
        pltpu.sync_copy(dst_hbm.at[pl.ds(base, ew)], edst)
        pltpu.sync_copy(src_hbm.at[pl.ds(base, ew)], esrc)

        iota = lax.iota(jnp.int32, 16)
        zero = jnp.zeros((16,), jnp.float32)

        def issue1(b, slot):
            for u in range(2):
                dstg[slot, pl.ds(u * 16, 16)] = edst[pl.ds(b * 32 + u * 16, 16)]
                sstg[slot, pl.ds(u * 16, 16)] = esrc[pl.ds(b * 32 + u * 16, 16)]
            pltpu.async_copy(q_hbm.at[dstg.at[slot]], qrows.at[slot], qsem.at[slot])
            pltpu.async_copy(k_hbm.at[sstg.at[slot]], krows.at[slot], ksem.at[slot])

        issue1(0, 0)

        def blk1(b, carry):
            slot = lax.rem(b, 2)
            nslot = 1 - slot

            @pl.when(b + 1 < nblk)
            def _():
                issue1(b + 1, nslot)

            pltpu.make_async_copy(q_hbm.at[dstg.at[slot]], qrows.at[slot],
                                  qsem.at[slot]).wait()
            pltpu.make_async_copy(k_hbm.at[sstg.at[slot]], krows.at[slot],
                                  ksem.at[slot]).wait()

            for half in range(2):
                def dot(i, r):
                    e = half * 16 + i
                    acc = zero
                    for j in range(NBB):
                        p = (qrows[slot, e, pl.ds(j * 32, 32)] *
                             krows[slot, e, pl.ds(j * 32, 32)])
                        u0, u1 = plsc.unpack(p, format=plsc.PackFormat.INTERLEAVED)
                        acc = acc + u0 + u1
                    return jnp.where(iota == i, jnp.sum(acc), r)

                r = lax.fori_loop(0, 16, dot, zero)
                eidx = base + b * 32 + half * 16 + iota
                wloc[pl.ds(b * 32 + half * 16, 16)] = jnp.where(
                    eidx < ecat, jnp.exp(r), 0.0)
            return carry

        lax.fori_loop(0, nblk, blk1, 0)

        zvb = jnp.zeros((32,), jnp.bfloat16)

        def zrow(rr_, c):
            def zcol(j, c2):
                zbuf[rr_, pl.ds(j * 32, 32)] = zvb
                return c2
            return lax.fori_loop(0, NBB, zcol, c)

        lax.fori_loop(0, ZR, zrow, 0)
        myrow = sid * RS

        zi = jnp.zeros((16,), jnp.int32)

        for p in range(npasses):
            lo = p * R
            for z in range(RS // ZR):
                pltpu.sync_copy(zbuf, accsh.at[pl.ds(myrow + z * ZR, ZR)])
            plsc.subcore_barrier()

            def scan(v, mcnt):
                d = edst[pl.ds(v * 16, 16)]
                m = (d >= lo) & (d < lo + R)
                plsc.store_compressed(mdl.at[pl.ds(mcnt, 16)], d - lo, mask=m)
                plsc.store_compressed(msrc.at[pl.ds(mcnt, 16)],
                                      esrc[pl.ds(v * 16, 16)], mask=m)
                plsc.store_compressed(mw.at[pl.ds(mcnt, 16)],
                                      wloc[pl.ds(v * 16, 16)], mask=m)
                cnt = plsc.all_reduce_population_count(m)
                if cnt.ndim:
                    cnt = cnt[0]
                return mcnt + cnt

            mcnt = lax.fori_loop(0, ew // 16, scan, jnp.int32(0))

            for u in range(2):
                mdl[pl.ds(mcnt + u * 16, 16)] = zi
                msrc[pl.ds(mcnt + u * 16, 16)] = zi
                mw[pl.ds(mcnt + u * 16, 16)] = zero
            nmb = (mcnt + 31) // 32

            def issue2(b, slot):
                for u in range(2):
                    sstg[slot, pl.ds(u * 16, 16)] = msrc[pl.ds(b * 32 + u * 16, 16)]
                    dstg[slot, pl.ds(u * 16, 16)] = mdl[pl.ds(b * 32 + u * 16, 16)]
                pltpu.async_copy(v_hbm.at[sstg.at[slot]], qrows.at[slot],
                                 qsem.at[slot])

            @pl.when(nmb > 0)
            def _():
                issue2(0, 0)

            def blk2(b, carry):
                slot = lax.rem(b, 2)
                nslot = 1 - slot
                pltpu.make_async_copy(v_hbm.at[sstg.at[slot]], qrows.at[slot],
                                      qsem.at[slot]).wait()

                @pl.when(b + 1 < nmb)
                def _():
                    issue2(b + 1, nslot)

                def scale(i, c):
                    wsc = plsc.load_gather(
                        mw, [jnp.full((16,), b * 32 + i, dtype=jnp.int32)])
                    wb = plsc.pack(wsc, wsc, format=plsc.PackFormat.INTERLEAVED)
                    for j in range(NBB):
                        qrows[slot, i, pl.ds(j * 32, 32)] = (
                            qrows[slot, i, pl.ds(j * 32, 32)] * wb)
                    return c

                lax.fori_loop(0, 32, scale, 0)
                pltpu.sync_copy(qrows.at[slot], accsh.at[dstg.at[slot]], add=True)
                return carry

            lax.fori_loop(0, nmb, blk2, 0)
            plsc.subcore_barrier()
            pltpu.sync_copy(accsh.at[pl.ds(myrow, RS)],
                            out_hbm.at[cc, pl.ds(lo + myrow, RS)])
            plsc.subcore_barrier()

    return s12


def _padw(w, k, m):
    return jnp.pad(w, ((0, k - w.shape[0]), (0, m - w.shape[1])))


def _padv(b, m):
    return jnp.pad(b, (0, m - b.shape[0]))


def _to_sc(x, nrows=None):
    n = x.shape[0] if nrows is None else nrows
    return jnp.pad(x, ((0, n - x.shape[0]), (0, HPB - HP))).astype(jnp.bfloat16)


def _edge_stage(q, kcat, vcat, src, dst, n_dst, e_cat):
    npad = _ceil_mult(n_dst, R)
    epad = _ceil_mult(e_cat, 1024)
    qb = _to_sc(q, npad)
    kb = _to_sc(kcat)
    vb = _to_sc(vcat)
    srcp = jnp.pad(src, (0, epad - e_cat))
    dstp = jnp.pad(dst, (0, epad - e_cat))
    return _make_edge(npad, kcat.shape[0], epad, e_cat)(qb, kb, vb, dstp, srcp)


def kernel(params, x_author, x_paper, edge_index_writes, edge_index_rev_writes,
           edge_index_cites):
    na, np_ = x_author.shape[0], x_paper.shape[0]
    types = {"author": x_author, "paper": x_paper}
    h = {}
    for t, x in types.items():
        lw = params["lin"][t]
        h[t] = _mm(x, _padw(lw["W"], x.shape[1], HP), _padv(lw["b"], HP), act="relu")

    sw, dw = edge_index_writes[0], edge_index_writes[1]
    srw, drw = edge_index_rev_writes[0], edge_index_rev_writes[1]
    sc_, dc_ = edge_index_cites[0], edge_index_cites[1]

    for lp in params["layers"]:
        k, q, v = {}, {}, {}
        for t in ("author", "paper"):
            wkqv = jnp.concatenate(
                [_padw(lp[nm][t]["W"], HP, HP) for nm in ("k", "q", "v")], axis=1)
            bkqv = jnp.concatenate(
                [_padv(lp[nm][t]["b"], HP) for nm in ("k", "q", "v")])
            kqv = _mm(h[t], wkqv, bkqv)
            k[t] = kqv[:, :HP]
            q[t] = kqv[:, HP:2 * HP]
            v[t] = kqv[:, 2 * HP:]

        zb = jnp.zeros((HP,), jnp.float32)
        rw = lp["rel"]["writes"]
        kw = _mm(k["author"], _padw(rw["k"] * (SCALE * rw["p"]), HP, HP), zb)
        vw = _mm(v["author"], _padw(rw["v"], HP, HP), zb, denom_mod=True)
        rr, rc = lp["rel"]["rev_writes"], lp["rel"]["cites"]
        wk2 = jnp.concatenate([_padw(rr["k"] * (SCALE * rr["p"]), HP, HP),
                               _padw(rc["k"] * (SCALE * rc["p"]), HP, HP)], axis=1)
        wv2 = jnp.concatenate([_padw(rr["v"], HP, HP),
                               _padw(rc["v"], HP, HP)], axis=1)
        kp2 = _mm(k["paper"], wk2, jnp.zeros((2 * HP,), jnp.float32))
        vp2 = _mm(v["paper"], wv2, jnp.zeros((2 * HP,), jnp.float32),
                  denom_mod=True)
        k_rev, k_cit = kp2[:, :HP], kp2[:, HP:]
        v_rev, v_cit = vp2[:, :HP], vp2[:, HP:]

        acc_p = _edge_stage(
            q["paper"],
            jnp.concatenate([kw, k_cit], axis=0),
            jnp.concatenate([vw, v_cit], axis=0),
            jnp.concatenate([sw, sc_ + na]),
            jnp.concatenate([dw, dc_]),
            np_, 2 * sw.shape[0])
        acc_a = _edge_stage(q["author"], k_rev, v_rev, srw, drw,
                            na, srw.shape[0])

        newh = {}
        for t, acc, n in (("paper", acc_p, np_), ("author", acc_a, na)):
            ow = lp["out"][t]
            sk = jax.nn.sigmoid(lp["skip"][t]).astype(jnp.float32)
            newh[t] = _out_stage(acc[0], acc[1], h[t],
                                 _padw(ow["W"], HPB, HP), _padv(ow["b"], HP), sk)
        h = newh

    return (h["author"][:, :H], h["paper"][:, :H])


# speedup vs baseline: 1.0245x; 1.0245x over previous
import functools
import math

import jax
import jax.numpy as jnp
from jax import lax
from jax.experimental import pallas as pl
from jax.experimental.pallas import tpu as pltpu
from jax.experimental.pallas import tpu_sc as plsc

H = 769
HP = 784
HPB = 800
NBB = HPB // 32
SCALE = 1.0 / math.sqrt(H)
NC, NS = 2, 16
NW = NC * NS
R = 2560
RS = R // NS
ZR = 16

_SC_PARAMS = pltpu.CompilerParams(use_tc_tiling_on_sc=False,
                                  needs_layout_passes=False)


def _ceil_mult(x, m):
    return ((x + m - 1) // m) * m


def _mm(a, w, b, act="none", denom_mod=False):
    n, k = a.shape
    m = w.shape[1]
    bn = 512
    grid = (pl.cdiv(n, bn),)

    def body(a_ref, w_ref, b_ref, o_ref):
        acc = jnp.dot(a_ref[...].astype(jnp.bfloat16),
                      w_ref[...].astype(jnp.bfloat16),
                      preferred_element_type=jnp.float32)
        acc = acc + b_ref[...]
        if act == "relu":
            acc = jnp.maximum(acc, 0.0)
        if denom_mod:
            col = lax.broadcasted_iota(jnp.int32, acc.shape, 1)
            acc = jnp.where(col % HP == H, 1.0, acc)
        o_ref[...] = acc

    return pl.pallas_call(
        body,
        grid=grid,
        in_specs=[
            pl.BlockSpec((bn, k), lambda i: (i, 0)),
            pl.BlockSpec((k, m), lambda i: (0, 0)),
            pl.BlockSpec((1, m), lambda i: (0, 0)),
        ],
        out_specs=pl.BlockSpec((bn, m), lambda i: (i, 0)),
        out_shape=jax.ShapeDtypeStruct((n, m), jnp.float32),
    )(a, w, b.reshape(1, m))


def _out_stage(acc0, acc1, hprev, w, b, sk):
    n = hprev.shape[0]
    bn = 512
    grid = (pl.cdiv(n, bn),)

    def body(a0_ref, a1_ref, h_ref, w_ref, b_ref, sk_ref, o_ref):
        s = a0_ref[...].astype(jnp.float32) + a1_ref[...].astype(jnp.float32)
        denom = s[:, H:H + 1]
        g = s * (1.0 / (denom + 1e-16))
        g = g * 0.5 * (1.0 + lax.erf(g * (1.0 / math.sqrt(2.0))))
        o = jnp.dot(g.astype(jnp.bfloat16), w_ref[...].astype(jnp.bfloat16),
                    preferred_element_type=jnp.float32) + b_ref[...]
        skv = sk_ref[0, 0]
        o_ref[...] = jnp.maximum(skv * o + (1.0 - skv) * h_ref[...], 0.0)

    return pl.pallas_call(
        body,
        grid=grid,
        in_specs=[
            pl.BlockSpec((bn, HPB), lambda i: (i, 0)),
            pl.BlockSpec((bn, HPB), lambda i: (i, 0)),
            pl.BlockSpec((bn, HP), lambda i: (i, 0)),
            pl.BlockSpec((HPB, HP), lambda i: (0, 0)),
            pl.BlockSpec((1, HP), lambda i: (0, 0)),
            pl.BlockSpec((1, 1), lambda i: (0, 0)),
        ],
        out_specs=pl.BlockSpec((bn, HP), lambda i: (i, 0)),
        out_shape=jax.ShapeDtypeStruct((n, HP), jnp.float32),
    )(acc0, acc1, hprev, w, b.reshape(1, HP), sk.reshape(1, 1))


@functools.lru_cache(maxsize=None)
def _make_s1(npad, nk, epad, ecat):
    ew = epad // NW
    nblk = ew // 64
    mesh = plsc.VectorSubcoreMesh(core_axis_name="c", subcore_axis_name="s",
                                  num_cores=NC, num_subcores=NS)

    @functools.partial(
        pl.kernel,
        out_type=jax.ShapeDtypeStruct((epad,), jnp.float32),
        mesh=mesh,
        compiler_params=_SC_PARAMS,
        scratch_types=[
            pltpu.VMEM((ew,), jnp.int32),
            pltpu.VMEM((ew,), jnp.int32),
            pltpu.VMEM((ew,), jnp.float32),
            pltpu.VMEM((2, 64), jnp.int32),
            pltpu.VMEM((2, 64), jnp.int32),
            pltpu.VMEM((2, 64, HPB), jnp.bfloat16),
            pltpu.VMEM((2, 64, HPB), jnp.bfloat16),
            pltpu.SemaphoreType.DMA((2,)),
            pltpu.SemaphoreType.DMA((2,)),
        ],
    )
    def s1(q_hbm, k_hbm, dst_hbm, src_hbm, w_hbm,
           edst, esrc, wloc, dstg, sstg, qrows, krows, qsem, ksem):
        wid = lax.axis_index("s") * NC + lax.axis_index("c")
        base = wid * ew
        pltpu.sync_copy(dst_hbm.at[pl.ds(base, ew)], edst)
        pltpu.sync_copy(src_hbm.at[pl.ds(base, ew)], esrc)

        def issue(b, slot):
            for u in range(4):
                dstg[slot, pl.ds(u * 16, 16)] = edst[pl.ds(b * 64 + u * 16, 16)]
                sstg[slot, pl.ds(u * 16, 16)] = esrc[pl.ds(b * 64 + u * 16, 16)]
            pltpu.async_copy(q_hbm.at[dstg.at[slot]], qrows.at[slot], qsem.at[slot])
            pltpu.async_copy(k_hbm.at[sstg.at[slot]], krows.at[slot], ksem.at[slot])

        issue(0, 0)
        iota = lax.iota(jnp.int32, 16)
        zero = jnp.zeros((16,), jnp.float32)

        def blk(b, carry):
            slot = lax.rem(b, 2)
            nslot = 1 - slot

            @pl.when(b + 1 < nblk)
            def _():
                issue(b + 1, nslot)

            pltpu.make_async_copy(q_hbm.at[dstg.at[slot]], qrows.at[slot],
                                  qsem.at[slot]).wait()
            pltpu.make_async_copy(k_hbm.at[sstg.at[slot]], krows.at[slot],
                                  ksem.at[slot]).wait()

            for quarter in range(4):
                def dot(i, r):
                    e = quarter * 16 + i
                    acc = zero
                    for j in range(NBB):
                        p = (qrows[slot, e, pl.ds(j * 32, 32)] *
                             krows[slot, e, pl.ds(j * 32, 32)])
                        u0, u1 = plsc.unpack(p, format=plsc.PackFormat.INTERLEAVED)
                        acc = acc + u0 + u1
                    return jnp.where(iota == i, jnp.sum(acc), r)

                r = lax.fori_loop(0, 16, dot, zero)
                eidx = base + b * 64 + quarter * 16 + iota
                wloc[pl.ds(b * 64 + quarter * 16, 16)] = jnp.where(
                    eidx < ecat, jnp.exp(r), 0.0)
            return carry

        lax.fori_loop(0, nblk, blk, 0)
        pltpu.sync_copy(wloc, w_hbm.at[pl.ds(base, ew)])

    return s1


@functools.lru_cache(maxsize=None)
def _make_s2(npad, nk, epad):
    ew = epad // NW
    npasses = npad // R
    mcap = ew + 48
    mesh = plsc.VectorSubcoreMesh(core_axis_name="c", subcore_axis_name="s",
                                  num_cores=NC, num_subcores=NS)

    @functools.partial(
        pl.kernel,
        out_type=jax.ShapeDtypeStruct((NC, npad, HPB), jnp.bfloat16),
        mesh=mesh,
        compiler_params=_SC_PARAMS,
        scratch_types=[
            pltpu.VMEM((ew,), jnp.int32),
            pltpu.VMEM((ew,), jnp.int32),
            pltpu.VMEM((ew,), jnp.float32),
            pltpu.VMEM((mcap,), jnp.int32),
            pltpu.VMEM((mcap,), jnp.int32),
            pltpu.VMEM((mcap,), jnp.float32),
            pltpu.VMEM((2, 32), jnp.int32),
            pltpu.VMEM((2, 32), jnp.int32),
            pltpu.VMEM((2, 32, HPB), jnp.bfloat16),
            pltpu.VMEM((ZR, HPB), jnp.bfloat16),
            pltpu.VMEM_SHARED((R, HPB), jnp.bfloat16),
            pltpu.SemaphoreType.DMA((2,)),
        ],
    )
    def s2(v_hbm, dst_hbm, src_hbm, w_hbm, out_hbm,
           edst, esrc, ewt, mdl, msrc, mw, sstg, istg, rows, zbuf, accsh, vsem):
        cc = lax.axis_index("c")
        sid = lax.axis_index("s")
        wid = sid * NC + cc
        base = wid * ew
        pltpu.sync_copy(dst_hbm.at[pl.ds(base, ew)], edst)
        pltpu.sync_copy(src_hbm.at[pl.ds(base, ew)], esrc)
        pltpu.sync_copy(w_hbm.at[pl.ds(base, ew)], ewt)

        zvb = jnp.zeros((32,), jnp.bfloat16)

        def zrow(r, c):
            def zcol(j, c2):
                zbuf[r, pl.ds(j * 32, 32)] = zvb
                return c2
            return lax.fori_loop(0, NBB, zcol, c)

        lax.fori_loop(0, ZR, zrow, 0)
        myrow = sid * RS

        zi = jnp.zeros((16,), jnp.int32)
        zv = jnp.zeros((16,), jnp.float32)

        for p in range(npasses):
            lo = p * R
            for z in range(RS // ZR):
                pltpu.sync_copy(zbuf, accsh.at[pl.ds(myrow + z * ZR, ZR)])
            plsc.subcore_barrier()

            def scan(v, mcnt):
                d = edst[pl.ds(v * 16, 16)]
                m = (d >= lo) & (d < lo + R)
                plsc.store_compressed(mdl.at[pl.ds(mcnt, 16)], d - lo, mask=m)
                plsc.store_compressed(msrc.at[pl.ds(mcnt, 16)],
                                      esrc[pl.ds(v * 16, 16)], mask=m)
                plsc.store_compressed(mw.at[pl.ds(mcnt, 16)],
                                      ewt[pl.ds(v * 16, 16)], mask=m)
                cnt = plsc.all_reduce_population_count(m)
                if cnt.ndim:
                    cnt = cnt[0]
                return mcnt + cnt

            mcnt = lax.fori_loop(0, ew // 16, scan, jnp.int32(0))

            for u in range(2):
                mdl[pl.ds(mcnt + u * 16, 16)] = zi
                msrc[pl.ds(mcnt + u * 16, 16)] = zi
                mw[pl.ds(mcnt + u * 16, 16)] = zv
            nmb = (mcnt + 31) // 32

            def issue(b, slot):
                for u in range(2):
                    sstg[slot, pl.ds(u * 16, 16)] = msrc[pl.ds(b * 32 + u * 16, 16)]
                    istg[slot, pl.ds(u * 16, 16)] = mdl[pl.ds(b * 32 + u * 16, 16)]
                pltpu.async_copy(v_hbm.at[sstg.at[slot]], rows.at[slot],
                                 vsem.at[slot])

            @pl.when(nmb > 0)
            def _():
                issue(0, 0)

            def blk(b, carry):
                slot = lax.rem(b, 2)
                nslot = 1 - slot
                pltpu.make_async_copy(v_hbm.at[sstg.at[slot]], rows.at[slot],
                                      vsem.at[slot]).wait()

                @pl.when(b + 1 < nmb)
                def _():
                    issue(b + 1, nslot)

                def scale(i, c):
                    wsc = plsc.load_gather(
                        mw, [jnp.full((16,), b * 32 + i, dtype=jnp.int32)])
                    wb = plsc.pack(wsc, wsc, format=plsc.PackFormat.INTERLEAVED)
                    for j in range(NBB):
                        rows[slot, i, pl.ds(j * 32, 32)] = (
                            rows[slot, i, pl.ds(j * 32, 32)] * wb)
                    return c

                lax.fori_loop(0, 32, scale, 0)
                pltpu.sync_copy(rows.at[slot], accsh.at[istg.at[slot]], add=True)
                return carry

            lax.fori_loop(0, nmb, blk, 0)
            plsc.subcore_barrier()
            pltpu.sync_copy(accsh.at[pl.ds(myrow, RS)],
                            out_hbm.at[cc, pl.ds(lo + myrow, RS)])
            plsc.subcore_barrier()

    return s2


def _padw(w, k, m):
    return jnp.pad(w, ((0, k - w.shape[0]), (0, m - w.shape[1])))


def _padv(b, m):
    return jnp.pad(b, (0, m - b.shape[0]))


def _to_sc(x, nrows=None):
    n = x.shape[0] if nrows is None else nrows
    return jnp.pad(x, ((0, n - x.shape[0]), (0, HPB - HP))).astype(jnp.bfloat16)


def _edge_stage(q, kcat, vcat, src, dst, n_dst, e_cat):
    npad = _ceil_mult(n_dst, R)
    epad = _ceil_mult(e_cat, 2048)
    qb = _to_sc(q, npad)
    kb = _to_sc(kcat)
    vb = _to_sc(vcat)
    srcp = jnp.pad(src, (0, epad - e_cat))
    dstp = jnp.pad(dst, (0, epad - e_cat))
    w = _make_s1(npad, kcat.shape[0], epad, e_cat)(qb, kb, dstp, srcp)
    acc = _make_s2(npad, vcat.shape[0], epad)(vb, dstp, srcp, w)
    return acc


def kernel(params, x_author, x_paper, edge_index_writes, edge_index_rev_writes,
           edge_index_cites):
    na, np_ = x_author.shape[0], x_paper.shape[0]
    types = {"author": x_author, "paper": x_paper}
    h = {}
    for t, x in types.items():
        lw = params["lin"][t]
        h[t] = _mm(x, _padw(lw["W"], x.shape[1], HP), _padv(lw["b"], HP), act="relu")

    sw, dw = edge_index_writes[0], edge_index_writes[1]
    srw, drw = edge_index_rev_writes[0], edge_index_rev_writes[1]
    sc_, dc_ = edge_index_cites[0], edge_index_cites[1]

    for lp in params["layers"]:
        k, q, v = {}, {}, {}
        for t in ("author", "paper"):
            wkqv = jnp.concatenate(
                [_padw(lp[nm][t]["W"], HP, HP) for nm in ("k", "q", "v")], axis=1)
            bkqv = jnp.concatenate(
                [_padv(lp[nm][t]["b"], HP) for nm in ("k", "q", "v")])
            kqv = _mm(h[t], wkqv, bkqv)
            k[t] = kqv[:, :HP]
            q[t] = kqv[:, HP:2 * HP]
            v[t] = kqv[:, 2 * HP:]

        zb = jnp.zeros((HP,), jnp.float32)
        rw = lp["rel"]["writes"]
        kw = _mm(k["author"], _padw(rw["k"] * (SCALE * rw["p"]), HP, HP), zb)
        vw = _mm(v["author"], _padw(rw["v"], HP, HP), zb, denom_mod=True)
        rr, rc = lp["rel"]["rev_writes"], lp["rel"]["cites"]
        wk2 = jnp.concatenate([_padw(rr["k"] * (SCALE * rr["p"]), HP, HP),
                               _padw(rc["k"] * (SCALE * rc["p"]), HP, HP)], axis=1)
        wv2 = jnp.concatenate([_padw(rr["v"], HP, HP),
                               _padw(rc["v"], HP, HP)], axis=1)
        kp2 = _mm(k["paper"], wk2, jnp.zeros((2 * HP,), jnp.float32))
        vp2 = _mm(v["paper"], wv2, jnp.zeros((2 * HP,), jnp.float32),
                  denom_mod=True)
        k_rev, k_cit = kp2[:, :HP], kp2[:, HP:]
        v_rev, v_cit = vp2[:, :HP], vp2[:, HP:]

        acc_p = _edge_stage(
            q["paper"],
            jnp.concatenate([kw, k_cit], axis=0),
            jnp.concatenate([vw, v_cit], axis=0),
            jnp.concatenate([sw, sc_ + na]),
            jnp.concatenate([dw, dc_]),
            np_, 2 * sw.shape[0])
        acc_a = _edge_stage(q["author"], k_rev, v_rev, srw, drw,
                            na, srw.shape[0])

        newh = {}
        for t, acc, n in (("paper", acc_p, np_), ("author", acc_a, na)):
            ow = lp["out"][t]
            sk = jax.nn.sigmoid(lp["skip"][t]).astype(jnp.float32)
            newh[t] = _out_stage(acc[0], acc[1], h[t],
                                 _padw(ow["W"], HPB, HP), _padv(ow["b"], HP), sk)
        h = newh

    return (h["author"][:, :H], h["paper"][:, :H])


# speedup vs baseline: 1.1139x; 1.0873x over previous
import functools
import math

import jax
import jax.numpy as jnp
from jax import lax
from jax.experimental import pallas as pl
from jax.experimental.pallas import tpu as pltpu
from jax.experimental.pallas import tpu_sc as plsc

H = 769
HP = 784
HPB = 800
NBB = HPB // 32
SCALE = 1.0 / math.sqrt(H)
NC, NS = 2, 16
NW = NC * NS
R = 2560
RS = R // NS
ZR = 8

_SC_PARAMS = pltpu.CompilerParams(use_tc_tiling_on_sc=False,
                                  needs_layout_passes=False)


def _ceil_mult(x, m):
    return ((x + m - 1) // m) * m


def _mm(a, w, b, act="none", denom_mod=False):
    n, k = a.shape
    m = w.shape[1]
    bn = 512
    grid = (pl.cdiv(n, bn),)

    def body(a_ref, w_ref, b_ref, o_ref):
        acc = jnp.dot(a_ref[...].astype(jnp.bfloat16),
                      w_ref[...].astype(jnp.bfloat16),
                      preferred_element_type=jnp.float32)
        acc = acc + b_ref[...]
        if act == "relu":
            acc = jnp.maximum(acc, 0.0)
        if denom_mod:
            col = lax.broadcasted_iota(jnp.int32, acc.shape, 1)
            acc = jnp.where(col % HP == H, 1.0, acc)
        o_ref[...] = acc

    return pl.pallas_call(
        body,
        grid=grid,
        in_specs=[
            pl.BlockSpec((bn, k), lambda i: (i, 0)),
            pl.BlockSpec((k, m), lambda i: (0, 0)),
            pl.BlockSpec((1, m), lambda i: (0, 0)),
        ],
        out_specs=pl.BlockSpec((bn, m), lambda i: (i, 0)),
        out_shape=jax.ShapeDtypeStruct((n, m), jnp.float32),
    )(a, w, b.reshape(1, m))


def _out_stage(acc0, hprev, w, b, sk):
    n = hprev.shape[0]
    bn = 512
    grid = (pl.cdiv(n, bn),)

    def body(a0_ref, h_ref, w_ref, b_ref, sk_ref, o_ref):
        s = a0_ref[...].astype(jnp.float32)
        denom = s[:, H:H + 1]
        g = s * (1.0 / (denom + 1e-16))
        g = g * 0.5 * (1.0 + lax.erf(g * (1.0 / math.sqrt(2.0))))
        o = jnp.dot(g.astype(jnp.bfloat16), w_ref[...].astype(jnp.bfloat16),
                    preferred_element_type=jnp.float32) + b_ref[...]
        skv = sk_ref[0, 0]
        o_ref[...] = jnp.maximum(skv * o + (1.0 - skv) * h_ref[...], 0.0)

    return pl.pallas_call(
        body,
        grid=grid,
        in_specs=[
            pl.BlockSpec((bn, HPB), lambda i: (i, 0)),
            pl.BlockSpec((bn, HP), lambda i: (i, 0)),
            pl.BlockSpec((HPB, HP), lambda i: (0, 0)),
            pl.BlockSpec((1, HP), lambda i: (0, 0)),
            pl.BlockSpec((1, 1), lambda i: (0, 0)),
        ],
        out_specs=pl.BlockSpec((bn, HP), lambda i: (i, 0)),
        out_shape=jax.ShapeDtypeStruct((n, HP), jnp.float32),
    )(acc0, hprev, w, b.reshape(1, HP), sk.reshape(1, 1))


@functools.lru_cache(maxsize=None)
def _make_s1(npad, nk, epad, ecat):
    ew = epad // NW
    nblk = ew // 64
    mesh = plsc.VectorSubcoreMesh(core_axis_name="c", subcore_axis_name="s",
                                  num_cores=NC, num_subcores=NS)

    @functools.partial(
        pl.kernel,
        out_type=jax.ShapeDtypeStruct((epad,), jnp.float32),
        mesh=mesh,
        compiler_params=_SC_PARAMS,
        scratch_types=[
            pltpu.VMEM((ew,), jnp.int32),
            pltpu.VMEM((ew,), jnp.int32),
            pltpu.VMEM((ew,), jnp.float32),
            pltpu.VMEM((2, 64), jnp.int32),
            pltpu.VMEM((2, 64), jnp.int32),
            pltpu.VMEM((2, 64, HPB), jnp.bfloat16),
            pltpu.VMEM((2, 64, HPB), jnp.bfloat16),
            pltpu.SemaphoreType.DMA((2,)),
            pltpu.SemaphoreType.DMA((2,)),
        ],
    )
    def s1(q_hbm, k_hbm, dst_hbm, src_hbm, w_hbm,
           edst, esrc, wloc, dstg, sstg, qrows, krows, qsem, ksem):
        wid = lax.axis_index("s") * NC + lax.axis_index("c")
        base = wid * ew
        pltpu.sync_copy(dst_hbm.at[pl.ds(base, ew)], edst)
        pltpu.sync_copy(src_hbm.at[pl.ds(base, ew)], esrc)

        def issue(b, slot):
            for u in range(4):
                dstg[slot, pl.ds(u * 16, 16)] = edst[pl.ds(b * 64 + u * 16, 16)]
                sstg[slot, pl.ds(u * 16, 16)] = esrc[pl.ds(b * 64 + u * 16, 16)]
            pltpu.async_copy(q_hbm.at[dstg.at[slot]], qrows.at[slot], qsem.at[slot])
            pltpu.async_copy(k_hbm.at[sstg.at[slot]], krows.at[slot], ksem.at[slot])

        issue(0, 0)
        iota = lax.iota(jnp.int32, 16)
        zero = jnp.zeros((16,), jnp.float32)

        def blk(b, carry):
            slot = lax.rem(b, 2)
            nslot = 1 - slot

            @pl.when(b + 1 < nblk)
            def _():
                issue(b + 1, nslot)

            pltpu.make_async_copy(q_hbm.at[dstg.at[slot]], qrows.at[slot],
                                  qsem.at[slot]).wait()
            pltpu.make_async_copy(k_hbm.at[sstg.at[slot]], krows.at[slot],
                                  ksem.at[slot]).wait()

            for quarter in range(4):
                def dot(i, r):
                    e = quarter * 16 + i
                    acc = zero
                    for j in range(NBB):
                        p = (qrows[slot, e, pl.ds(j * 32, 32)] *
                             krows[slot, e, pl.ds(j * 32, 32)])
                        u0, u1 = plsc.unpack(p, format=plsc.PackFormat.INTERLEAVED)
                        acc = acc + u0 + u1
                    return jnp.where(iota == i, jnp.sum(acc), r)

                r = lax.fori_loop(0, 16, dot, zero)
                eidx = base + b * 64 + quarter * 16 + iota
                wloc[pl.ds(b * 64 + quarter * 16, 16)] = jnp.where(
                    eidx < ecat, jnp.exp(r), 0.0)
            return carry

        lax.fori_loop(0, nblk, blk, 0)
        pltpu.sync_copy(wloc, w_hbm.at[pl.ds(base, ew)])

    return s1


@functools.lru_cache(maxsize=None)
def _make_s2(npad, nk, epad):
    ew = epad // NS
    npasses = npad // R
    mcap = ew + 48
    mesh = plsc.VectorSubcoreMesh(core_axis_name="c", subcore_axis_name="s",
                                  num_cores=NC, num_subcores=NS)

    @functools.partial(
        pl.kernel,
        out_type=jax.ShapeDtypeStruct((npad, HPB), jnp.bfloat16),
        mesh=mesh,
        compiler_params=_SC_PARAMS,
        scratch_types=[
            pltpu.VMEM((ew,), jnp.int32),
            pltpu.VMEM((ew,), jnp.int32),
            pltpu.VMEM((ew,), jnp.float32),
            pltpu.VMEM((mcap,), jnp.int32),
            pltpu.VMEM((mcap,), jnp.int32),
            pltpu.VMEM((mcap,), jnp.float32),
            pltpu.VMEM((2, 32), jnp.int32),
            pltpu.VMEM((2, 32), jnp.int32),
            pltpu.VMEM((2, 32, HPB), jnp.bfloat16),
            pltpu.VMEM((ZR, HPB), jnp.bfloat16),
            pltpu.VMEM_SHARED((R, HPB), jnp.bfloat16),
            pltpu.SemaphoreType.DMA((2,)),
        ],
    )
    def s2(v_hbm, dst_hbm, src_hbm, w_hbm, out_hbm,
           edst, esrc, ewt, mdl, msrc, mw, sstg, istg, rows, zbuf, accsh, vsem):
        cc = lax.axis_index("c")
        sid = lax.axis_index("s")
        base = sid * ew
        pltpu.sync_copy(dst_hbm.at[pl.ds(base, ew)], edst)
        pltpu.sync_copy(src_hbm.at[pl.ds(base, ew)], esrc)
        pltpu.sync_copy(w_hbm.at[pl.ds(base, ew)], ewt)

        zvb = jnp.zeros((32,), jnp.bfloat16)

        def zrow(r, c):
            def zcol(j, c2):
                zbuf[r, pl.ds(j * 32, 32)] = zvb
                return c2
            return lax.fori_loop(0, NBB, zcol, c)

        lax.fori_loop(0, ZR, zrow, 0)
        myrow = sid * RS

        zi = jnp.zeros((16,), jnp.int32)
        zv = jnp.zeros((16,), jnp.float32)

        for pp in range(npasses // NC):
            lo = (pp * NC + cc) * R
            for z in range(RS // ZR):
                pltpu.sync_copy(zbuf, accsh.at[pl.ds(myrow + z * ZR, ZR)])
            plsc.subcore_barrier()

            def scan(v, mcnt):
                d = edst[pl.ds(v * 16, 16)]
                m = (d >= lo) & (d < lo + R)
                plsc.store_compressed(mdl.at[pl.ds(mcnt, 16)], d - lo, mask=m)
                plsc.store_compressed(msrc.at[pl.ds(mcnt, 16)],
                                      esrc[pl.ds(v * 16, 16)], mask=m)
                plsc.store_compressed(mw.at[pl.ds(mcnt, 16)],
                                      ewt[pl.ds(v * 16, 16)], mask=m)
                cnt = plsc.all_reduce_population_count(m)
                if cnt.ndim:
                    cnt = cnt[0]
                return mcnt + cnt

            mcnt = lax.fori_loop(0, ew // 16, scan, jnp.int32(0))

            for u in range(2):
                mdl[pl.ds(mcnt + u * 16, 16)] = zi
                msrc[pl.ds(mcnt + u * 16, 16)] = zi
                mw[pl.ds(mcnt + u * 16, 16)] = zv
            nmb = (mcnt + 31) // 32

            def issue(b, slot):
                for u in range(2):
                    sstg[slot, pl.ds(u * 16, 16)] = msrc[pl.ds(b * 32 + u * 16, 16)]
                    istg[slot, pl.ds(u * 16, 16)] = mdl[pl.ds(b * 32 + u * 16, 16)]
                pltpu.async_copy(v_hbm.at[sstg.at[slot]], rows.at[slot],
                                 vsem.at[slot])

            @pl.when(nmb > 0)
            def _():
                issue(0, 0)

            def blk(b, carry):
                slot = lax.rem(b, 2)
                nslot = 1 - slot
                pltpu.make_async_copy(v_hbm.at[sstg.at[slot]], rows.at[slot],
                                      vsem.at[slot]).wait()

                @pl.when(b + 1 < nmb)
                def _():
                    issue(b + 1, nslot)

                def scale(i, c):
                    wsc = plsc.load_gather(
                        mw, [jnp.full((16,), b * 32 + i, dtype=jnp.int32)])
                    wb = plsc.pack(wsc, wsc, format=plsc.PackFormat.INTERLEAVED)
                    for j in range(NBB):
                        rows[slot, i, pl.ds(j * 32, 32)] = (
                            rows[slot, i, pl.ds(j * 32, 32)] * wb)
                    return c

                lax.fori_loop(0, 32, scale, 0)
                pltpu.sync_copy(rows.at[slot], accsh.at[istg.at[slot]], add=True)
                return carry

            lax.fori_loop(0, nmb, blk, 0)
            plsc.subcore_barrier()
            pltpu.sync_copy(accsh.at[pl.ds(myrow, RS)],
                            out_hbm.at[pl.ds(lo + myrow, RS)])
            plsc.subcore_barrier()

    return s2


def _padw(w, k, m):
    return jnp.pad(w, ((0, k - w.shape[0]), (0, m - w.shape[1])))


def _padv(b, m):
    return jnp.pad(b, (0, m - b.shape[0]))


def _to_sc(x, nrows=None):
    n = x.shape[0] if nrows is None else nrows
    return jnp.pad(x, ((0, n - x.shape[0]), (0, HPB - HP))).astype(jnp.bfloat16)


def _edge_stage(q, kcat, vcat, src, dst, n_dst, e_cat):
    npad = _ceil_mult(n_dst, NC * R)
    epad = _ceil_mult(e_cat, 2048)
    qb = _to_sc(q, npad)
    kb = _to_sc(kcat)
    vb = _to_sc(vcat)
    srcp = jnp.pad(src, (0, epad - e_cat))
    dstp = jnp.pad(dst, (0, epad - e_cat))
    w = _make_s1(npad, kcat.shape[0], epad, e_cat)(qb, kb, dstp, srcp)
    acc = _make_s2(npad, vcat.shape[0], epad)(vb, dstp, srcp, w)
    return acc


def kernel(params, x_author, x_paper, edge_index_writes, edge_index_rev_writes,
           edge_index_cites):
    na, np_ = x_author.shape[0], x_paper.shape[0]
    types = {"author": x_author, "paper": x_paper}
    h = {}
    for t, x in types.items():
        lw = params["lin"][t]
        h[t] = _mm(x, _padw(lw["W"], x.shape[1], HP), _padv(lw["b"], HP), act="relu")

    sw, dw = edge_index_writes[0], edge_index_writes[1]
    srw, drw = edge_index_rev_writes[0], edge_index_rev_writes[1]
    sc_, dc_ = edge_index_cites[0], edge_index_cites[1]

    for lp in params["layers"]:
        k, q, v = {}, {}, {}
        for t in ("author", "paper"):
            wkqv = jnp.concatenate(
                [_padw(lp[nm][t]["W"], HP, HP) for nm in ("k", "q", "v")], axis=1)
            bkqv = jnp.concatenate(
                [_padv(lp[nm][t]["b"], HP) for nm in ("k", "q", "v")])
            kqv = _mm(h[t], wkqv, bkqv)
            k[t] = kqv[:, :HP]
            q[t] = kqv[:, HP:2 * HP]
            v[t] = kqv[:, 2 * HP:]

        zb = jnp.zeros((HP,), jnp.float32)
        rw = lp["rel"]["writes"]
        kw = _mm(k["author"], _padw(rw["k"] * (SCALE * rw["p"]), HP, HP), zb)
        vw = _mm(v["author"], _padw(rw["v"], HP, HP), zb, denom_mod=True)
        rr, rc = lp["rel"]["rev_writes"], lp["rel"]["cites"]
        wk2 = jnp.concatenate([_padw(rr["k"] * (SCALE * rr["p"]), HP, HP),
                               _padw(rc["k"] * (SCALE * rc["p"]), HP, HP)], axis=1)
        wv2 = jnp.concatenate([_padw(rr["v"], HP, HP),
                               _padw(rc["v"], HP, HP)], axis=1)
        kp2 = _mm(k["paper"], wk2, jnp.zeros((2 * HP,), jnp.float32))
        vp2 = _mm(v["paper"], wv2, jnp.zeros((2 * HP,), jnp.float32),
                  denom_mod=True)
        k_rev, k_cit = kp2[:, :HP], kp2[:, HP:]
        v_rev, v_cit = vp2[:, :HP], vp2[:, HP:]

        acc_p = _edge_stage(
            q["paper"],
            jnp.concatenate([kw, k_cit], axis=0),
            jnp.concatenate([vw, v_cit], axis=0),
            jnp.concatenate([sw, sc_ + na]),
            jnp.concatenate([dw, dc_]),
            np_, 2 * sw.shape[0])
        acc_a = _edge_stage(q["author"], k_rev, v_rev, srw, drw,
                            na, srw.shape[0])

        newh = {}
        for t, acc, n in (("paper", acc_p, np_), ("author", acc_a, na)):
            ow = lp["out"][t]
            sk = jax.nn.sigmoid(lp["skip"][t]).astype(jnp.float32)
            newh[t] = _out_stage(acc, h[t],
                                 _padw(ow["W"], HPB, HP), _padv(ow["b"], HP), sk)
        h = newh

    return (h["author"][:, :H], h["paper"][:, :H])
